# Initial kernel scaffold; baseline (speedup 1.0000x reference)
import jax
import jax.numpy as jnp
from jax.experimental import pallas as pl


def kernel(x, edge_index, adj_values, W1, b1, W2, b2):
    raise NotImplementedError("write your pallas kernel here")


# speedup vs baseline: 2.6361x; 2.6361x over previous
import functools

import jax
import jax.numpy as jnp
from jax import lax
from jax.experimental import pallas as pl
from jax.experimental.pallas import tpu as pltpu
from jax.experimental.pallas import tpu_sc as plsc

_N = 10000
_E = 320000
_D = 128
_ALPHA = 0.1
_K = 10

_N_PAD = 10240
_NSUB = 16
_CHUNK = 128
_CH = 157
_E_PAD = _NSUB * _CH * _CHUNK
_JG = 4
_ROWS_PER_SUB = _N_PAD // _NSUB
_UP_CHUNK = 64
_UP_STEPS = _ROWS_PER_SUB // _UP_CHUNK


def _mlp_body(x_ref, w1_ref, b1_ref, w2_ref, b2_ref, o_ref):
    h = jnp.dot(x_ref[...], w1_ref[...], preferred_element_type=jnp.float32)
    h = jnp.maximum(h + b1_ref[...], 0.0)
    o_ref[...] = (
        jnp.dot(h, w2_ref[...], preferred_element_type=jnp.float32) + b2_ref[...]
    )


def _mlp(x_pad, W1, b1, W2, b2):
    blk = 1024
    return pl.pallas_call(
        _mlp_body,
        grid=(_N_PAD // blk,),
        in_specs=[
            pl.BlockSpec((blk, _D), lambda i: (i, 0)),
            pl.BlockSpec((_D, _D), lambda i: (0, 0)),
            pl.BlockSpec((1, _D), lambda i: (0, 0)),
            pl.BlockSpec((_D, _D), lambda i: (0, 0)),
            pl.BlockSpec((1, _D), lambda i: (0, 0)),
        ],
        out_specs=pl.BlockSpec((blk, _D), lambda i: (i, 0)),
        out_shape=jax.ShapeDtypeStruct((_N_PAD, _D), jnp.float32),
    )(x_pad, W1, b1.reshape(1, _D), W2, b2.reshape(1, _D))


def _sc_propagate(hL, hR, src_r, dst_r, val_r):
    mesh = plsc.VectorSubcoreMesh(core_axis_name="c", subcore_axis_name="s")
    out_t = [jax.ShapeDtypeStruct((_N_PAD, 64), jnp.float32)] * 2
    scratch = [
        pltpu.VMEM((_CH, _CHUNK), jnp.int32),
        pltpu.VMEM((_CH, _CHUNK), jnp.int32),
        pltpu.VMEM((_CH, _CHUNK), jnp.float32),
        pltpu.VMEM((_CHUNK, 64), jnp.float32),
        pltpu.VMEM((_UP_CHUNK, 64), jnp.float32),
        pltpu.VMEM((_UP_CHUNK, 64), jnp.float32),
        pltpu.VMEM((_UP_CHUNK, 64), jnp.float32),
        pltpu.VMEM_SHARED((_N_PAD, 64), jnp.float32),
        pltpu.SemaphoreType.DMA,
    ]

    @functools.partial(
        pl.kernel, out_type=out_t, scratch_types=scratch, mesh=mesh,
        compiler_params=pltpu.CompilerParams(use_tc_tiling_on_sc=False))
    def k(hL_h, hR_h, src_h, dst_h, val_h, zL_h, zR_h,
          src_v, dst_v, val_v, gbuf, abuf, zbuf, zzero, agg, gsem):
        c = lax.axis_index("c")
        s = lax.axis_index("s")
        pltpu.sync_copy(src_h.at[s], src_v)
        pltpu.sync_copy(dst_h.at[s], dst_v)
        pltpu.sync_copy(val_h.at[s], val_v)
        for i in range(_UP_CHUNK):
            for j in range(_JG):
                zzero[i, pl.ds(16 * j, 16)] = jnp.zeros((16,), jnp.float32)

        def run(h_hbm, z_hbm):
            base = s * _ROWS_PER_SUB

            def init_step(t, carry):
                r0 = base + t * _UP_CHUNK
                pltpu.sync_copy(h_hbm.at[pl.ds(r0, _UP_CHUNK)], zbuf)
                pltpu.sync_copy(zbuf, z_hbm.at[pl.ds(r0, _UP_CHUNK)])
                pltpu.sync_copy(zzero, agg.at[pl.ds(r0, _UP_CHUNK)])
                return carry

            lax.fori_loop(0, _UP_STEPS, init_step, 0)
            plsc.subcore_barrier()

            def k_step(kk, carry):
                def chunk_step(jj, carry2):
                    pltpu.async_copy(z_hbm.at[src_v.at[jj]], gbuf, gsem).wait()

                    def edge_group(g, carry3):
                        v16 = val_v[jj, pl.ds(g * 16, 16)]
                        for u in range(16):
                            e = g * 16 + u
                            v = lax.gather(
                                v16,
                                jnp.full((16, 1), u, jnp.int32),
                                lax.GatherDimensionNumbers(
                                    offset_dims=(),
                                    collapsed_slice_dims=(0,),
                                    start_index_map=(0,),
                                ),
                                slice_sizes=(1,),
                                mode=lax.GatherScatterMode.PROMISE_IN_BOUNDS,
                            )
                            for j in range(_JG):
                                gbuf[e, pl.ds(16 * j, 16)] = (
                                    gbuf[e, pl.ds(16 * j, 16)] * v)
                        return carry3

                    lax.fori_loop(0, _CHUNK // 16, edge_group, 0)
                    pltpu.sync_copy(gbuf, agg.at[dst_v.at[jj]], add=True)
                    return carry2

                lax.fori_loop(0, _CH, chunk_step, 0)
                plsc.subcore_barrier()

                def up_step(t, carry2):
                    r0 = base + t * _UP_CHUNK
                    pltpu.sync_copy(agg.at[pl.ds(r0, _UP_CHUNK)], abuf)
                    pltpu.sync_copy(zzero, agg.at[pl.ds(r0, _UP_CHUNK)])
                    pltpu.sync_copy(z_hbm.at[pl.ds(r0, _UP_CHUNK)], zbuf)

                    def row_step(i, carry3):
                        for j in range(_JG):
                            sl = pl.ds(16 * j, 16)
                            zbuf[i, sl] = (
                                (1.0 - _ALPHA) * abuf[i, sl]
                                + _ALPHA * zbuf[i, sl]
                            )
                        return carry3

                    lax.fori_loop(0, _UP_CHUNK, row_step, 0)
                    pltpu.sync_copy(zbuf, z_hbm.at[pl.ds(r0, _UP_CHUNK)])
                    return carry2

                lax.fori_loop(0, _UP_STEPS, up_step, 0)
                plsc.subcore_barrier()
                return carry

            lax.fori_loop(0, _K, k_step, 0)

        @pl.when(c == 0)
        def _():
            run(hL_h, zL_h)

        @pl.when(c == 1)
        def _():
            run(hR_h, zR_h)

    return k(hL, hR, src_r, dst_r, val_r)


def kernel(x, edge_index, adj_values, W1, b1, W2, b2):
    x_pad = jnp.zeros((_N_PAD, _D), jnp.float32).at[:_N].set(x)
    h = _mlp(x_pad, W1, b1, W2, b2)
    h = h.at[_N:].set(0.0)

    scale = 1.0 / (1.0 - 0.0 + 1e-05)
    vals = adj_values.astype(jnp.float32) * scale
    dst = edge_index[0].astype(jnp.int32)
    src = edge_index[1].astype(jnp.int32)
    pad = _E_PAD - _E
    dst_p = jnp.concatenate([dst, jnp.full((pad,), _N_PAD - 1, jnp.int32)])
    src_p = jnp.concatenate([src, jnp.full((pad,), _N_PAD - 1, jnp.int32)])
    val_p = jnp.concatenate([vals, jnp.zeros((pad,), jnp.float32)])
    src_r = src_p.reshape(_NSUB, _CH, _CHUNK)
    dst_r = dst_p.reshape(_NSUB, _CH, _CHUNK)
    val_r = val_p.reshape(_NSUB, _CH, _CHUNK)

    hL = h[:, :64]
    hR = h[:, 64:]
    zL, zR = _sc_propagate(hL, hR, src_r, dst_r, val_r)
    out = jnp.concatenate([zL, zR], axis=1)
    return out[:_N]
